# baseline (speedup 1.0000x reference)
import functools

import jax
import jax.numpy as jnp
from jax import lax
from jax.experimental import pallas as pl
from jax.experimental.pallas import tpu as pltpu
from jax.experimental.pallas import tpu_sc as plsc

N_TOKENS = 16384
NUM_EXP = 8
NUM_CORES = 1
NUM_SUBCORES = 8
NUM_WORKERS = NUM_CORES * NUM_SUBCORES
W_PER_EXP = NUM_WORKERS // NUM_EXP
CHUNK = N_TOKENS // W_PER_EXP
LANES = 16
VREGS_PER_CHUNK = CHUNK // LANES

_mesh = plsc.VectorSubcoreMesh(core_axis_name="c", subcore_axis_name="s", num_cores=1, num_subcores=8)


@functools.partial(
    pl.kernel,
    mesh=_mesh,
    out_type=jax.ShapeDtypeStruct((NUM_EXP, N_TOKENS), jnp.float32),
    scratch_types=[
        pltpu.VMEM((CHUNK,), jnp.float32),
        pltpu.VMEM((LANES,), jnp.float32),
        pltpu.SemaphoreType.DMA,
        pltpu.SemaphoreType.DMA,
        pltpu.SemaphoreType.DMA,
    ],
)
def _masked_routing_sc(rt_hbm, m_hbm, out_hbm, buf, mask_vmem, m_sem, d_sem,
                       o_sem):
    wid = lax.axis_index("s") * NUM_CORES + lax.axis_index("c")
    exp = wid // W_PER_EXP
    base = (wid % W_PER_EXP) * CHUNK
    half = CHUNK // 2
    m0 = pltpu.async_copy(m_hbm, mask_vmem.at[pl.ds(0, NUM_EXP)], m_sem)
    m1 = pltpu.async_copy(m_hbm, mask_vmem.at[pl.ds(NUM_EXP, NUM_EXP)], m_sem)
    d0 = pltpu.async_copy(rt_hbm.at[exp, pl.ds(base, half)],
                          buf.at[pl.ds(0, half)], d_sem)
    d1 = pltpu.async_copy(rt_hbm.at[exp, pl.ds(base + half, half)],
                          buf.at[pl.ds(half, half)], d_sem)
    m0.wait()
    m1.wait()
    mask = mask_vmem[...].at[jnp.full((LANES,), exp, dtype=jnp.int32)].get(
        mode="promise_in_bounds")
    d0.wait()
    d1.wait()

    unroll = 8

    def body(i, carry):
        for j in range(unroll):
            sl = pl.ds(i * (LANES * unroll) + j * LANES, LANES)
            buf[sl] = buf[sl] * mask
        return carry

    steps = VREGS_PER_CHUNK // unroll
    lax.fori_loop(0, steps // 2, body, 0)
    o0 = pltpu.async_copy(buf.at[pl.ds(0, half)],
                          out_hbm.at[exp, pl.ds(base, half)], o_sem)
    lax.fori_loop(steps // 2, steps, body, 0)
    o1 = pltpu.async_copy(buf.at[pl.ds(half, half)],
                          out_hbm.at[exp, pl.ds(base + half, half)], o_sem)
    o0.wait()
    o1.wait()


def kernel(x, router_outputs, expert_masks):
    del x
    out_t = _masked_routing_sc(router_outputs.T, expert_masks)
    return out_t.T


# speedup vs baseline: 1.0387x; 1.0387x over previous
import functools

import jax
import jax.numpy as jnp
from jax import lax
from jax.experimental import pallas as pl
from jax.experimental.pallas import tpu as pltpu
from jax.experimental.pallas import tpu_sc as plsc

N_TOKENS = 16384
NUM_EXP = 8
NUM_CORES = 1
NUM_SUBCORES = 16
NUM_WORKERS = NUM_CORES * NUM_SUBCORES
W_PER_EXP = NUM_WORKERS // NUM_EXP
CHUNK = N_TOKENS // W_PER_EXP
LANES = 16
VREGS_PER_CHUNK = CHUNK // LANES

_mesh = plsc.VectorSubcoreMesh(core_axis_name="c", subcore_axis_name="s", num_cores=1)


@functools.partial(
    pl.kernel,
    mesh=_mesh,
    out_type=jax.ShapeDtypeStruct((NUM_EXP, N_TOKENS), jnp.float32),
    scratch_types=[
        pltpu.VMEM((CHUNK,), jnp.float32),
        pltpu.VMEM((LANES,), jnp.float32),
        pltpu.SemaphoreType.DMA,
        pltpu.SemaphoreType.DMA,
        pltpu.SemaphoreType.DMA,
    ],
)
def _masked_routing_sc(rt_hbm, m_hbm, out_hbm, buf, mask_vmem, m_sem, d_sem,
                       o_sem):
    wid = lax.axis_index("s") * NUM_CORES + lax.axis_index("c")
    exp = wid // W_PER_EXP
    base = (wid % W_PER_EXP) * CHUNK
    nq = 4
    quarter = CHUNK // nq
    m0 = pltpu.async_copy(m_hbm, mask_vmem.at[pl.ds(0, NUM_EXP)], m_sem)
    m1 = pltpu.async_copy(m_hbm, mask_vmem.at[pl.ds(NUM_EXP, NUM_EXP)], m_sem)
    din = [
        pltpu.async_copy(rt_hbm.at[exp, pl.ds(base + q * quarter, quarter)],
                         buf.at[pl.ds(q * quarter, quarter)], d_sem)
        for q in range(nq)
    ]
    m0.wait()
    m1.wait()
    mask = mask_vmem[...].at[jnp.full((LANES,), exp, dtype=jnp.int32)].get(
        mode="promise_in_bounds")

    unroll = 8

    def body(i, carry):
        for j in range(unroll):
            sl = pl.ds(i * (LANES * unroll) + j * LANES, LANES)
            buf[sl] = buf[sl] * mask
        return carry

    steps_per_q = quarter // (LANES * unroll)
    dout = []
    for q in range(nq):
        din[q].wait()
        lax.fori_loop(q * steps_per_q, (q + 1) * steps_per_q, body, 0)
        dout.append(
            pltpu.async_copy(buf.at[pl.ds(q * quarter, quarter)],
                             out_hbm.at[exp, pl.ds(base + q * quarter, quarter)],
                             o_sem))
    for c in dout:
        c.wait()


def kernel(x, router_outputs, expert_masks):
    del x
    out_t = _masked_routing_sc(router_outputs.T, expert_masks)
    return out_t.T


# speedup vs baseline: 1.0426x; 1.0038x over previous
import functools

import jax
import jax.numpy as jnp
from jax import lax
from jax.experimental import pallas as pl
from jax.experimental.pallas import tpu as pltpu
from jax.experimental.pallas import tpu_sc as plsc

N_TOKENS = 16384
NUM_EXP = 8
NUM_CORES = 1
NUM_SUBCORES = 16
NUM_WORKERS = NUM_CORES * NUM_SUBCORES
W_PER_EXP = NUM_WORKERS // NUM_EXP
CHUNK = N_TOKENS // W_PER_EXP
LANES = 16
VREGS_PER_CHUNK = CHUNK // LANES

_mesh = plsc.VectorSubcoreMesh(core_axis_name="c", subcore_axis_name="s", num_cores=1)


@functools.partial(
    pl.kernel,
    mesh=_mesh,
    out_type=jax.ShapeDtypeStruct((NUM_EXP, N_TOKENS), jnp.float32),
    scratch_types=[
        pltpu.VMEM((CHUNK,), jnp.float32),
        pltpu.VMEM((LANES,), jnp.float32),
        pltpu.SemaphoreType.DMA,
        pltpu.SemaphoreType.DMA,
        pltpu.SemaphoreType.DMA,
    ],
)
def _masked_routing_sc(rt_hbm, m_hbm, out_hbm, buf, mask_vmem, m_sem, d_sem,
                       o_sem):
    wid = lax.axis_index("s") * NUM_CORES + lax.axis_index("c")
    exp = wid // W_PER_EXP
    base = (wid % W_PER_EXP) * CHUNK
    nq = 4
    quarter = CHUNK // nq
    m0 = pltpu.async_copy(m_hbm, mask_vmem.at[pl.ds(0, NUM_EXP)], m_sem)
    m1 = pltpu.async_copy(m_hbm, mask_vmem.at[pl.ds(NUM_EXP, NUM_EXP)], m_sem)
    din = [
        pltpu.async_copy(rt_hbm.at[exp, pl.ds(base + q * quarter, quarter)],
                         buf.at[pl.ds(q * quarter, quarter)], d_sem)
        for q in range(nq)
    ]
    m0.wait()
    m1.wait()
    mask = mask_vmem[...].at[jnp.full((LANES,), exp, dtype=jnp.int32)].get(
        mode="promise_in_bounds")

    dout = []
    for q in range(nq):
        din[q].wait()

        @plsc.parallel_loop(q * quarter, (q + 1) * quarter, step=LANES,
                            unroll=8)
        def _scale(i):
            buf[pl.ds(i, LANES)] = buf[pl.ds(i, LANES)] * mask

        dout.append(
            pltpu.async_copy(buf.at[pl.ds(q * quarter, quarter)],
                             out_hbm.at[exp, pl.ds(base + q * quarter, quarter)],
                             o_sem))
    for c in dout:
        c.wait()


def kernel(x, router_outputs, expert_masks):
    del x
    out_t = _masked_routing_sc(router_outputs.T, expert_masks)
    return out_t.T
